# baseline (speedup 1.0000x reference)
import functools

import jax
import jax.numpy as jnp
from jax import lax
from jax.experimental import pallas as pl
from jax.experimental.pallas import tpu as pltpu
from jax.experimental.pallas import tpu_sc as plsc

LABEL_SIZE = 1000
HIDDEN = 768
EPS = 1e-12

NUM_CORES = 2
NUM_SUBCORES = 16
NW = NUM_CORES * NUM_SUBCORES

ROW_CHUNKS = 8
CHUNK = 128
LAST_CHUNK = LABEL_SIZE - (ROW_CHUNKS - 1) * CHUNK
BATCH_GROUPS = NW // ROW_CHUNKS
WAVE = 16

TILE_B = 160
SHARED_ROWS = 4 * CHUNK


def _ln_body(w_ref, gamma_ref, beta_ref, out_ref):
    x = w_ref[...]
    mu = jnp.mean(x, axis=-1, keepdims=True)
    var = jnp.mean(jnp.square(x - mu), axis=-1, keepdims=True)
    out_ref[...] = (x - mu) * lax.rsqrt(var + EPS) * gamma_ref[...] + beta_ref[...]


def _layer_norm_table(W, gamma, beta):
    return pl.pallas_call(
        _ln_body,
        out_shape=jax.ShapeDtypeStruct((LABEL_SIZE, HIDDEN), jnp.float32),
    )(W, gamma, beta)


def _dual_stream(tile_src, sp_src, out_hbm, row0, nrows, tb0, tb_n, sb0, sb_n,
                 tsem, ssem):
    tile_waves = [(w0, min(WAVE, tb_n - w0)) for w0 in range(0, tb_n, WAVE)]
    sp_waves = [(w0, min(WAVE, sb_n - w0)) for w0 in range(0, sb_n, WAVE)]
    n = max(len(tile_waves), len(sp_waves))
    for i in range(n):
        copies = []
        if i < len(tile_waves):
            w0, nw = tile_waves[i]
            copies += [
                pltpu.async_copy(
                    tile_src, out_hbm.at[tb0 + w0 + j, pl.ds(row0, nrows), :],
                    tsem)
                for j in range(nw)
            ]
        if i < len(sp_waves):
            w0, nw = sp_waves[i]
            copies += [
                pltpu.async_copy(
                    sp_src, out_hbm.at[sb0 + w0 + j, pl.ds(row0, nrows), :],
                    ssem)
                for j in range(nw)
            ]
        for c in copies:
            c.wait()


def _bcast_body(batch, ln_hbm, out_hbm, buf, shared, tsem, ssem):
    tb_n = TILE_B // BATCH_GROUPS
    sb_n = (batch - TILE_B) // BATCH_GROUPS

    wid = lax.axis_index("s") * NUM_CORES + lax.axis_index("c")
    rc = wid % ROW_CHUNKS
    bg = wid // ROW_CHUNKS
    row0 = rc * CHUNK
    srow0 = (rc // NUM_CORES) * CHUNK
    tb0 = bg * tb_n
    sb0 = TILE_B + bg * sb_n

    @pl.when(rc < ROW_CHUNKS - 1)
    def _():
        pltpu.sync_copy(ln_hbm.at[pl.ds(row0, CHUNK), :], buf)

        @pl.when(bg == 0)
        def _():
            pltpu.sync_copy(buf, shared.at[pl.ds(srow0, CHUNK), :])

        plsc.subcore_barrier()
        _dual_stream(buf, shared.at[pl.ds(srow0, CHUNK), :], out_hbm,
                     row0, CHUNK, tb0, tb_n, sb0, sb_n, tsem, ssem)

    @pl.when(rc == ROW_CHUNKS - 1)
    def _():
        small = buf.at[pl.ds(0, LAST_CHUNK), :]
        pltpu.sync_copy(ln_hbm.at[pl.ds(row0, LAST_CHUNK), :], small)

        @pl.when(bg == 0)
        def _():
            pltpu.sync_copy(small, shared.at[pl.ds(srow0, LAST_CHUNK), :])

        plsc.subcore_barrier()
        _dual_stream(small, shared.at[pl.ds(srow0, LAST_CHUNK), :], out_hbm,
                     row0, LAST_CHUNK, tb0, tb_n, sb0, sb_n, tsem, ssem)


def kernel(input_ids, W, gamma, beta):
    batch = input_ids.shape[0]
    assert TILE_B % BATCH_GROUPS == 0 and (batch - TILE_B) % BATCH_GROUPS == 0

    ln = _layer_norm_table(W, gamma, beta)

    mesh = plsc.VectorSubcoreMesh(core_axis_name="c", subcore_axis_name="s")
    bcast = functools.partial(
        pl.kernel,
        out_type=jax.ShapeDtypeStruct((batch, LABEL_SIZE, HIDDEN), jnp.float32),
        mesh=mesh,
        scratch_types=[
            pltpu.VMEM((CHUNK, HIDDEN), jnp.float32),
            pltpu.VMEM_SHARED((SHARED_ROWS, HIDDEN), jnp.float32),
            pltpu.SemaphoreType.DMA,
            pltpu.SemaphoreType.DMA,
        ],
    )(functools.partial(_bcast_body, batch))
    return bcast(ln)


# speedup vs baseline: 1.0594x; 1.0594x over previous
import functools

import jax
import jax.numpy as jnp
from jax import lax
from jax.experimental import pallas as pl
from jax.experimental.pallas import tpu as pltpu
from jax.experimental.pallas import tpu_sc as plsc

LABEL_SIZE = 1000
HIDDEN = 768
EPS = 1e-12

NUM_CORES = 2
NUM_SUBCORES = 16
NW = NUM_CORES * NUM_SUBCORES

GROUP_ROWS = 40
GROUPS = LABEL_SIZE // GROUP_ROWS
STAGE_GROUPS = 2
INNER = 20


def _ln_body(w_ref, gamma_ref, beta_ref, out_ref):
    x = w_ref[...]
    mu = jnp.mean(x, axis=-1, keepdims=True)
    var = jnp.mean(jnp.square(x - mu), axis=-1, keepdims=True)
    out_ref[...] = (x - mu) * lax.rsqrt(var + EPS) * gamma_ref[...] + beta_ref[...]


def _layer_norm_table(W, gamma, beta):
    return pl.pallas_call(
        _ln_body,
        out_shape=jax.ShapeDtypeStruct((LABEL_SIZE, HIDDEN), jnp.float32),
    )(W, gamma, beta)


def _bcast_body(batch, ln_hbm, out_hbm, buf, sem):
    tasks_per_w = GROUPS * batch // NW
    waves = tasks_per_w // INNER
    assert waves * INNER == tasks_per_w

    wid = lax.axis_index("s") * NUM_CORES + lax.axis_index("c")
    t0 = wid * tasks_per_w
    g0 = jnp.minimum(t0 // batch, GROUPS - STAGE_GROUPS)

    pltpu.sync_copy(
        ln_hbm.at[pl.ds(g0 * GROUP_ROWS, STAGE_GROUPS * GROUP_ROWS), :], buf
    )

    def wave(i, carry):
        base = t0 + i * INNER
        copies = []
        for j in range(INNER):
            t = base + j
            g = t // batch
            b = t % batch
            local = g - g0
            copies.append(
                pltpu.async_copy(
                    buf.at[pl.ds(local * GROUP_ROWS, GROUP_ROWS), :],
                    out_hbm.at[b, pl.ds(g * GROUP_ROWS, GROUP_ROWS), :],
                    sem,
                )
            )
        for c in copies:
            c.wait()
        return carry

    lax.fori_loop(0, waves, wave, None)


def kernel(input_ids, W, gamma, beta):
    batch = input_ids.shape[0]
    assert (GROUPS * batch) % (NW * INNER) == 0

    ln = _layer_norm_table(W, gamma, beta)

    mesh = plsc.VectorSubcoreMesh(core_axis_name="c", subcore_axis_name="s")
    bcast = functools.partial(
        pl.kernel,
        out_type=jax.ShapeDtypeStruct((batch, LABEL_SIZE, HIDDEN), jnp.float32),
        mesh=mesh,
        scratch_types=[
            pltpu.VMEM((STAGE_GROUPS * GROUP_ROWS, HIDDEN), jnp.float32),
            pltpu.SemaphoreType.DMA,
        ],
    )(functools.partial(_bcast_body, batch))
    return bcast(ln)
